# baseline (speedup 1.0000x reference)
import functools

import jax
import jax.numpy as jnp
from jax import lax
from jax.experimental import pallas as pl
from jax.experimental.pallas import tpu as pltpu
from jax.experimental.pallas import tpu_sc as plsc

BATCH = 4
SEQ_LEN = 2048
D_MODEL = 768

_NUM_CORES = 2
_NUM_SUBCORES = 16
_NW = _NUM_CORES * _NUM_SUBCORES
_S_PER_W = SEQ_LEN // _NW
_HALF = _S_PER_W // 2
_NHC = BATCH * 2
_LANES = 16
_D_SLICES = D_MODEL // _LANES


def _body(x_hbm, tok_hbm, pos_hbm, out_hbm, idx_v, pos_v, tok0, tok1, sems):
    wid = lax.axis_index("s") * _NUM_CORES + lax.axis_index("c")
    s_base = wid * _S_PER_W
    toks = (tok0, tok1)

    for b in range(BATCH):
        pltpu.sync_copy(x_hbm.at[b, pl.ds(s_base, _S_PER_W)], idx_v.at[b])

    def start_gather(i):
        b, h = divmod(i, 2)
        idx = idx_v.at[b, pl.ds(h * _HALF, _HALF)]
        return pltpu.async_copy(tok_hbm.at[idx], toks[i % 2], sems[i % 2])

    def start_store(i):
        b, h = divmod(i, 2)
        dst = out_hbm.at[b, pl.ds(s_base + h * _HALF, _HALF)]
        return pltpu.async_copy(toks[i % 2], dst, sems[2 + i % 2])

    gathers = [None] * _NHC
    stores = [None] * _NHC
    gathers[0] = start_gather(0)
    pos_cp = pltpu.async_copy(pos_hbm.at[pl.ds(s_base, _S_PER_W)], pos_v,
                              sems[4])

    for i in range(_NHC):
        if i + 1 < _NHC:
            if i >= 1:
                stores[i - 1].wait()
            gathers[i + 1] = start_gather(i + 1)
        gathers[i].wait()
        if i == 0:
            pos_cp.wait()

        h = i % 2
        buf = toks[i % 2]

        def per_row(r, _):
            for j in range(_D_SLICES):
                sl = pl.ds(j * _LANES, _LANES)
                buf[r, sl] = buf[r, sl] + pos_v[r + h * _HALF, sl]
            return 0

        lax.fori_loop(0, _HALF, per_row, 0, unroll=False)
        stores[i] = start_store(i)

    stores[_NHC - 2].wait()
    stores[_NHC - 1].wait()


@functools.partial(
    pl.kernel,
    out_type=jax.ShapeDtypeStruct((BATCH, SEQ_LEN, D_MODEL), jnp.float32),
    mesh=plsc.VectorSubcoreMesh(core_axis_name="c", subcore_axis_name="s"),
    scratch_types=[
        pltpu.VMEM((BATCH, _S_PER_W), jnp.int32),
        pltpu.VMEM((_S_PER_W, D_MODEL), jnp.float32),
        pltpu.VMEM((_HALF, D_MODEL), jnp.float32),
        pltpu.VMEM((_HALF, D_MODEL), jnp.float32),
        [pltpu.SemaphoreType.DMA] * 5,
    ],
)
def _emb_lookup(x_hbm, tok_hbm, pos_hbm, out_hbm, idx_v, pos_v, tok0, tok1,
                sems):
    _body(x_hbm, tok_hbm, pos_hbm, out_hbm, idx_v, pos_v, tok0, tok1, sems)


def kernel(x, token_table, position_table):
    x = x.astype(jnp.int32)
    return _emb_lookup(x, token_table, position_table)


# speedup vs baseline: 1.2499x; 1.2499x over previous
import functools

import jax
import jax.numpy as jnp
from jax import lax
from jax.experimental import pallas as pl
from jax.experimental.pallas import tpu as pltpu
from jax.experimental.pallas import tpu_sc as plsc

BATCH = 4
SEQ_LEN = 2048
D_MODEL = 768

_NUM_CORES = 2
_NUM_SUBCORES = 16
_NW = _NUM_CORES * _NUM_SUBCORES
_S_PER_W = SEQ_LEN // _NW
_LANES = 16
_D_SLICES = D_MODEL // _LANES


def _body(x_hbm, tok_hbm, pos_hbm, out_hbm, idx_v, tok_v, pos_v, sem):
    wid = lax.axis_index("s") * _NUM_CORES + lax.axis_index("c")
    s_base = wid * _S_PER_W

    pltpu.sync_copy(pos_hbm.at[pl.ds(s_base, _S_PER_W)], pos_v)
    for b in range(BATCH):
        pltpu.sync_copy(x_hbm.at[b, pl.ds(s_base, _S_PER_W)], idx_v.at[b])

    def per_batch(b, _):
        pltpu.async_copy(tok_hbm.at[idx_v.at[b]], tok_v, sem).wait()

        def per_row(r, _):
            for j in range(_D_SLICES):
                sl = pl.ds(j * _LANES, _LANES)
                plsc.addupdate(tok_v.at[r, sl], pos_v[r, sl])
            return 0

        lax.fori_loop(0, _S_PER_W, per_row, 0, unroll=False)
        pltpu.sync_copy(tok_v, out_hbm.at[b, pl.ds(s_base, _S_PER_W)])
        return 0

    lax.fori_loop(0, BATCH, per_batch, 0, unroll=False)


@functools.partial(
    pl.kernel,
    out_type=jax.ShapeDtypeStruct((BATCH, SEQ_LEN, D_MODEL), jnp.float32),
    mesh=plsc.VectorSubcoreMesh(core_axis_name="c", subcore_axis_name="s"),
    scratch_types=[
        pltpu.VMEM((BATCH, _S_PER_W), jnp.int32),
        pltpu.VMEM((_S_PER_W, D_MODEL), jnp.float32),
        pltpu.VMEM((_S_PER_W, D_MODEL), jnp.float32),
        pltpu.SemaphoreType.DMA,
    ],
)
def _emb_lookup(x_hbm, tok_hbm, pos_hbm, out_hbm, idx_v, tok_v, pos_v, sem):
    _body(x_hbm, tok_hbm, pos_hbm, out_hbm, idx_v, tok_v, pos_v, sem)


def kernel(x, token_table, position_table):
    x = x.astype(jnp.int32)
    return _emb_lookup(x, token_table, position_table)


# speedup vs baseline: 1.3969x; 1.1176x over previous
import functools

import jax
import jax.numpy as jnp
from jax import lax
from jax.experimental import pallas as pl
from jax.experimental.pallas import tpu as pltpu
from jax.experimental.pallas import tpu_sc as plsc

BATCH = 4
SEQ_LEN = 2048
D_MODEL = 768
_ROWS = BATCH * SEQ_LEN

_NUM_CORES = 2
_NUM_SUBCORES = 16
_NW = _NUM_CORES * _NUM_SUBCORES
_S_PER_W = SEQ_LEN // _NW
_HALF = _S_PER_W // 2
_NHC = BATCH * 2
_LANES = 16
_D_SLICES = D_MODEL // _LANES


def _body(x_hbm, tok_hbm, pos_hbm, out_hbm, idx_v, tok_v, pos_v, sem):
    wid = lax.axis_index("s") * _NUM_CORES + lax.axis_index("c")
    s_base = wid * _S_PER_W

    pltpu.sync_copy(pos_hbm.at[pl.ds(s_base, _S_PER_W)], pos_v)
    for b in range(BATCH):
        pltpu.sync_copy(x_hbm.at[b, pl.ds(s_base, _S_PER_W)],
                        idx_v.at[pl.ds(b * _S_PER_W, _S_PER_W)])

    def gather(i):
        off = lax.rem(i, 2) * _HALF
        return pltpu.make_async_copy(
            tok_hbm.at[idx_v.at[pl.ds(i * _HALF, _HALF)]],
            tok_v.at[pl.ds(off, _HALF)], sem)

    def add_and_store(i):
        off = lax.rem(i, 2) * _HALF

        def per_row(r, _):
            for j in range(_D_SLICES):
                sl = pl.ds(j * _LANES, _LANES)
                plsc.addupdate(tok_v.at[off + r, sl], pos_v[off + r, sl])
            return 0

        lax.fori_loop(0, _HALF, per_row, 0, unroll=False)
        row_base = lax.div(i, 2) * SEQ_LEN + s_base + lax.rem(i, 2) * _HALF
        pltpu.sync_copy(tok_v.at[pl.ds(off, _HALF)],
                        out_hbm.at[pl.ds(row_base, _HALF)])

    gather(0).start()

    def step(i, _):
        gather(i).wait()
        gather(i + 1).start()
        add_and_store(i)
        return 0

    lax.fori_loop(0, _NHC - 1, step, 0, unroll=False)
    gather(_NHC - 1).wait()
    add_and_store(_NHC - 1)


@functools.partial(
    pl.kernel,
    out_type=jax.ShapeDtypeStruct((_ROWS, D_MODEL), jnp.float32),
    mesh=plsc.VectorSubcoreMesh(core_axis_name="c", subcore_axis_name="s"),
    scratch_types=[
        pltpu.VMEM((BATCH * _S_PER_W,), jnp.int32),
        pltpu.VMEM((_S_PER_W, D_MODEL), jnp.float32),
        pltpu.VMEM((_S_PER_W, D_MODEL), jnp.float32),
        pltpu.SemaphoreType.DMA,
    ],
)
def _emb_lookup(x_hbm, tok_hbm, pos_hbm, out_hbm, idx_v, tok_v, pos_v, sem):
    _body(x_hbm, tok_hbm, pos_hbm, out_hbm, idx_v, tok_v, pos_v, sem)


def kernel(x, token_table, position_table):
    x = x.astype(jnp.int32)
    out = _emb_lookup(x, token_table, position_table)
    return out.reshape(BATCH, SEQ_LEN, D_MODEL)
